# baseline (speedup 1.0000x reference)
import functools

import jax
import jax.numpy as jnp
from jax import lax
from jax.experimental import pallas as pl
from jax.experimental.pallas import tpu as pltpu
from jax.experimental.pallas import tpu_sc as plsc

_NC = 2
_NS = 16
_NW = _NC * _NS

_V = 1000000
_D = 32
_LANES = 128
_FULL = _V // _LANES
_TAIL = _V - _FULL * _LANES


@functools.cache
def _make_linearize():
    mesh = plsc.VectorSubcoreMesh(core_axis_name="c", subcore_axis_name="s")
    n_loop = _FULL // _NW
    n_rem = _FULL - n_loop * _NW

    @functools.partial(
        pl.kernel,
        mesh=mesh,
        out_type=jax.ShapeDtypeStruct((_V * _D,), jnp.float32),
        scratch_types=[
            pltpu.VMEM((4, 8, _LANES), jnp.float32),
            pltpu.VMEM((_LANES * _D,), jnp.float32),
            pltpu.VMEM((4, 8, _TAIL), jnp.float32),
            pltpu.VMEM((_TAIL * _D,), jnp.float32),
            pltpu.SemaphoreType.DMA,
        ],
        compiler_params=pltpu.CompilerParams(use_tc_tiling_on_sc=True,
                                             needs_layout_passes=False),
    )
    def linearize(tab_t_hbm, out_hbm, blk_v, out_v, blk_t, out_t, sem):
        wid = lax.axis_index("s") * _NC + lax.axis_index("c")
        iota = lax.iota(jnp.int32, 16)
        sub_lo = [iota >> 3, iota & 7]
        sub_hi = [(iota >> 3) + 2, iota & 7]

        def do_block(c, blk, outb, width):
            handles = [
                pltpu.async_copy(
                    tab_t_hbm.at[pl.ds(r * 8, 8), pl.ds(c * _LANES, width)],
                    blk.at[r], sem)
                for r in range(4)
            ]
            for hd in handles:
                hd.wait()

            def col(i, carry):
                isp = jnp.full((16,), i, dtype=jnp.int32)
                lo = plsc.load_gather(blk, sub_lo + [isp])
                hi = plsc.load_gather(blk, sub_hi + [isp])
                outb[pl.ds(i * _D, 16)] = lo
                outb[pl.ds(i * _D + 16, 16)] = hi
                return carry

            lax.fori_loop(0, width, col, 0)
            pltpu.sync_copy(outb, out_hbm.at[pl.ds(c * _LANES * _D,
                                                   width * _D)])

        def body(k, carry):
            do_block(wid + k * _NW, blk_v, out_v, _LANES)
            return carry

        lax.fori_loop(0, n_loop, body, 0)

        @pl.when(wid < n_rem)
        def _():
            do_block(n_loop * _NW + wid, blk_v, out_v, _LANES)

        @pl.when(wid == n_rem)
        def _():
            do_block(_FULL, blk_t, out_t, _TAIL)

    return linearize


@functools.cache
def _make_gather(b, h, d, nb):
    b_per_w = b // _NW
    n_chunks = b_per_w // nb
    mesh = plsc.VectorSubcoreMesh(core_axis_name="c", subcore_axis_name="s")

    @functools.partial(
        pl.kernel,
        mesh=mesh,
        out_type=jax.ShapeDtypeStruct((b, h, d), jnp.float32),
        scratch_types=[
            pltpu.VMEM((nb, h), jnp.int32),
            pltpu.VMEM((nb, h, d), jnp.float32),
            pltpu.SemaphoreType.DMA,
        ],
        compiler_params=pltpu.CompilerParams(use_tc_tiling_on_sc=False),
    )
    def gather(table_hbm, idx_hbm, out_hbm, idx_v, rows_v, sem):
        wid = lax.axis_index("s") * _NC + lax.axis_index("c")
        base = wid * b_per_w

        def body(i, carry):
            b0 = base + i * nb
            pltpu.sync_copy(idx_hbm.at[pl.ds(b0, nb)], idx_v)
            handles = [
                pltpu.async_copy(table_hbm.at[idx_v.at[r]], rows_v.at[r], sem)
                for r in range(nb)
            ]
            for hd in handles:
                hd.wait()
            pltpu.sync_copy(rows_v, out_hbm.at[pl.ds(b0, nb)])
            return carry

        lax.fori_loop(0, n_chunks, body, 0)

    return gather


def kernel(input, embedding_matrix):
    b, h = input.shape
    v, d = embedding_matrix.shape
    idx = input.astype(jnp.int32)
    t_lin = _make_linearize()(embedding_matrix.T)
    table_lin = t_lin.reshape(v, d)
    return _make_gather(b, h, d, 16)(table_lin, idx)


# speedup vs baseline: 1.6527x; 1.6527x over previous
import functools

import jax
import jax.numpy as jnp
from jax import lax
from jax.experimental import pallas as pl
from jax.experimental.pallas import tpu as pltpu
from jax.experimental.pallas import tpu_sc as plsc

_NC = 2
_NS = 16
_NW = _NC * _NS

_V = 1000000
_D = 32
_LANES = 128
_FULL = _V // _LANES
_TAIL = _V - _FULL * _LANES


@functools.cache
def _make_linearize():
    mesh = plsc.VectorSubcoreMesh(core_axis_name="c", subcore_axis_name="s")
    n_loop = _FULL // _NW
    n_rem = _FULL - n_loop * _NW

    @functools.partial(
        pl.kernel,
        mesh=mesh,
        out_type=jax.ShapeDtypeStruct((_V * _D,), jnp.float32),
        scratch_types=[
            pltpu.VMEM((4, 8, _LANES), jnp.float32),
            pltpu.VMEM((_LANES * _D,), jnp.float32),
            pltpu.VMEM((4, 8, _TAIL), jnp.float32),
            pltpu.VMEM((_TAIL * _D,), jnp.float32),
            pltpu.SemaphoreType.DMA,
        ],
        compiler_params=pltpu.CompilerParams(use_tc_tiling_on_sc=True,
                                             needs_layout_passes=False),
    )
    def linearize(tab_t_hbm, out_hbm, blk_v, out_v, blk_t, out_t, sem):
        wid = lax.axis_index("s") * _NC + lax.axis_index("c")
        iota = lax.iota(jnp.int32, 16)
        sub_lo = [iota >> 3, iota & 7]
        sub_hi = [(iota >> 3) + 2, iota & 7]

        def do_block(c, blk, outb, width):
            handles = [
                pltpu.async_copy(
                    tab_t_hbm.at[pl.ds(r * 8, 8), pl.ds(c * _LANES, width)],
                    blk.at[r], sem)
                for r in range(4)
            ]
            for hd in handles:
                hd.wait()

            @plsc.parallel_loop(0, width, 1, unroll=8)
            def _(t):
                idx_i = ((t >> 4) << 4) + ((iota + t) & 15)
                lo = plsc.load_gather(blk, sub_lo + [idx_i])
                plsc.store_scatter(outb, [idx_i * _D + iota], lo)
                hi = plsc.load_gather(blk, sub_hi + [idx_i])
                plsc.store_scatter(outb, [idx_i * _D + 16 + iota], hi)

            pltpu.sync_copy(outb, out_hbm.at[pl.ds(c * _LANES * _D,
                                                   width * _D)])

        def body(k, carry):
            do_block(wid + k * _NW, blk_v, out_v, _LANES)
            return carry

        lax.fori_loop(0, n_loop, body, 0)

        @pl.when(wid < n_rem)
        def _():
            do_block(n_loop * _NW + wid, blk_v, out_v, _LANES)

        @pl.when(wid == n_rem)
        def _():
            do_block(_FULL, blk_t, out_t, _TAIL)

    return linearize


@functools.cache
def _make_gather(b, h, d, nb):
    b_per_w = b // _NW
    n_chunks = b_per_w // nb
    mesh = plsc.VectorSubcoreMesh(core_axis_name="c", subcore_axis_name="s")

    @functools.partial(
        pl.kernel,
        mesh=mesh,
        out_type=jax.ShapeDtypeStruct((b, h, d), jnp.float32),
        scratch_types=[
            pltpu.VMEM((nb, h), jnp.int32),
            pltpu.VMEM((nb, h, d), jnp.float32),
            pltpu.SemaphoreType.DMA,
        ],
        compiler_params=pltpu.CompilerParams(use_tc_tiling_on_sc=False),
    )
    def gather(table_hbm, idx_hbm, out_hbm, idx_v, rows_v, sem):
        wid = lax.axis_index("s") * _NC + lax.axis_index("c")
        base = wid * b_per_w

        def body(i, carry):
            b0 = base + i * nb
            pltpu.sync_copy(idx_hbm.at[pl.ds(b0, nb)], idx_v)
            handles = [
                pltpu.async_copy(table_hbm.at[idx_v.at[r]], rows_v.at[r], sem)
                for r in range(nb)
            ]
            for hd in handles:
                hd.wait()
            pltpu.sync_copy(rows_v, out_hbm.at[pl.ds(b0, nb)])
            return carry

        lax.fori_loop(0, n_chunks, body, 0)

    return gather


def kernel(input, embedding_matrix):
    b, h = input.shape
    v, d = embedding_matrix.shape
    idx = input.astype(jnp.int32)
    t_lin = _make_linearize()(embedding_matrix.T)
    table_lin = t_lin.reshape(v, d)
    return _make_gather(b, h, d, 16)(table_lin, idx)
